# Initial kernel scaffold; baseline (speedup 1.0000x reference)
import jax
import jax.numpy as jnp
from jax.experimental import pallas as pl


def kernel(query, key, value):
    raise NotImplementedError("write your pallas kernel here")


# speedup vs baseline: 1.9987x; 1.9987x over previous
import functools
import math

import jax
import jax.numpy as jnp
from jax.experimental import pallas as pl
from jax.experimental.pallas import tpu as pltpu

_B, _H, _QL, _KL, _D = 8, 32, 4, 2048, 128
_T = 512


def _flash_body(q_ref, k_ref, v_ref, o_ref, acc_ref, m_ref, l_ref):
    c = pl.program_id(1)
    nc = pl.num_programs(1)

    @pl.when(c == 0)
    def _init():
        m_ref[...] = jnp.full_like(m_ref, -jnp.inf)
        l_ref[...] = jnp.zeros_like(l_ref)
        acc_ref[...] = jnp.zeros_like(acc_ref)

    k = k_ref[0]
    v = v_ref[0]
    q = q_ref[0]

    kmin = jnp.min(k, axis=0, keepdims=True)
    kmax = jnp.max(k, axis=0, keepdims=True)
    ks = (kmax - kmin) * (1.0 / 3.0)
    ks = jnp.where(ks == 0, 1.0, ks)
    kinv = 1.0 / ks
    kd = jnp.clip(jnp.round((k - kmin) * kinv), 0.0, 3.0) * ks + kmin

    vmin = jnp.min(v, axis=(0, 2), keepdims=True)
    vmax = jnp.max(v, axis=(0, 2), keepdims=True)
    vs = (vmax - vmin) * (1.0 / 15.0)
    vs = jnp.where(vs == 0, 1.0, vs)
    vinv = 1.0 / vs
    vd = jnp.clip(jnp.round((v - vmin) * vinv), 0.0, 15.0) * vs + vmin

    scale = 1.0 / math.sqrt(float(_D))
    s = jax.lax.dot_general(
        q, kd, (((2,), (2,)), ((0,), (0,))),
        preferred_element_type=jnp.float32,
    ) * scale

    m_prev = m_ref[...]
    m_new = jnp.maximum(m_prev, jnp.max(s, axis=2))
    alpha = jnp.exp(m_prev - m_new)
    p = jnp.exp(s - m_new[..., None])
    l_ref[...] = l_ref[...] * alpha + jnp.sum(p, axis=2)
    pv = jax.lax.dot_general(
        p, vd, (((2,), (1,)), ((0,), (0,))),
        preferred_element_type=jnp.float32,
    )
    acc_ref[...] = acc_ref[...] * alpha[..., None] + pv
    m_ref[...] = m_new

    @pl.when(c == nc - 1)
    def _flush():
        o_ref[0] = acc_ref[...] / l_ref[...][..., None]


@jax.jit
def kernel(query, key, value):
    nc = _KL // _T
    grid = (_B, nc)
    out = pl.pallas_call(
        _flash_body,
        grid=grid,
        in_specs=[
            pl.BlockSpec((1, _H, _QL, _D), lambda b, c: (b, 0, 0, 0)),
            pl.BlockSpec((1, _H, _T, _D), lambda b, c: (b, 0, c, 0)),
            pl.BlockSpec((1, _H, _T, _D), lambda b, c: (b, 0, c, 0)),
        ],
        out_specs=pl.BlockSpec((1, _H, _QL, _D), lambda b, c: (b, 0, 0, 0)),
        out_shape=jax.ShapeDtypeStruct((_B, _H, _QL, _D), jnp.float32),
        scratch_shapes=[
            pltpu.VMEM((_H, _QL, _D), jnp.float32),
            pltpu.VMEM((_H, _QL), jnp.float32),
            pltpu.VMEM((_H, _QL), jnp.float32),
        ],
        compiler_params=pltpu.CompilerParams(
            dimension_semantics=("parallel", "arbitrary"),
        ),
    )(query, key, value)
    return out


# speedup vs baseline: 2.0808x; 1.0411x over previous
import functools
import math

import jax
import jax.numpy as jnp
from jax.experimental import pallas as pl
from jax.experimental.pallas import tpu as pltpu

_B, _H, _QL, _KL, _D = 8, 32, 4, 2048, 128
_T = 512


def _flash_body(q_ref, k_ref, v_ref, o_ref, acc_ref, m_ref, l_ref):
    c = pl.program_id(1)
    nc = pl.num_programs(1)

    @pl.when(c == 0)
    def _init():
        m_ref[...] = jnp.full_like(m_ref, -jnp.inf)
        l_ref[...] = jnp.zeros_like(l_ref)
        acc_ref[...] = jnp.zeros_like(acc_ref)

    k = k_ref[0]
    v = v_ref[0]
    q = q_ref[0]

    kmin = jnp.min(k, axis=0, keepdims=True)
    kmax = jnp.max(k, axis=0, keepdims=True)
    ks = (kmax - kmin) * (1.0 / 3.0)
    ks = jnp.where(ks == 0, 1.0, ks)
    kinv = 1.0 / ks
    kb = -kmin * kinv
    ksq = jnp.round(k * kinv + kb) * ks

    vm0 = jnp.min(v, axis=0)
    vx0 = jnp.max(v, axis=0)
    vmin = jnp.min(vm0, axis=1)
    vmax = jnp.max(vx0, axis=1)
    vs = (vmax - vmin) * (1.0 / 15.0)
    vs = jnp.where(vs == 0, 1.0, vs)
    vinv = vs_inv = 1.0 / vs
    vb = -vmin * vinv
    vq = jnp.round(v * vinv[None, :, None] + vb[None, :, None])

    scale = 1.0 / math.sqrt(float(_D))
    s1 = jax.lax.dot_general(
        q, ksq, (((2,), (2,)), ((0,), (0,))),
        preferred_element_type=jnp.float32,
    )
    s2 = jax.lax.dot_general(
        q.reshape(_H * _QL, _D), kmin[0], (((1,), (1,)), ((), ())),
        preferred_element_type=jnp.float32,
    ).reshape(_H, _QL, _T)
    s = (s1 + s2) * scale

    m_prev = m_ref[...]
    m_new = jnp.maximum(m_prev, jnp.max(s, axis=2))
    alpha = jnp.exp(m_prev - m_new)
    p = jnp.exp(s - m_new[..., None])
    l_ref[...] = l_ref[...] * alpha + jnp.sum(p, axis=2)
    pv = jax.lax.dot_general(
        p * vs[None, None, :], vq, (((2,), (1,)), ((0,), (0,))),
        preferred_element_type=jnp.float32,
    )
    corr = jnp.sum(p * vmin[None, None, :], axis=2)
    acc_ref[...] = acc_ref[...] * alpha[..., None] + pv + corr[..., None]
    m_ref[...] = m_new

    @pl.when(c == nc - 1)
    def _flush():
        o_ref[0] = acc_ref[...] / l_ref[...][..., None]


@jax.jit
def kernel(query, key, value):
    nc = _KL // _T
    grid = (_B, nc)
    out = pl.pallas_call(
        _flash_body,
        grid=grid,
        in_specs=[
            pl.BlockSpec((1, _H, _QL, _D), lambda b, c: (b, 0, 0, 0)),
            pl.BlockSpec((1, _H, _T, _D), lambda b, c: (b, 0, c, 0)),
            pl.BlockSpec((1, _H, _T, _D), lambda b, c: (b, 0, c, 0)),
        ],
        out_specs=pl.BlockSpec((1, _H, _QL, _D), lambda b, c: (b, 0, 0, 0)),
        out_shape=jax.ShapeDtypeStruct((_B, _H, _QL, _D), jnp.float32),
        scratch_shapes=[
            pltpu.VMEM((_H, _QL, _D), jnp.float32),
            pltpu.VMEM((_H, _QL), jnp.float32),
            pltpu.VMEM((_H, _QL), jnp.float32),
        ],
        compiler_params=pltpu.CompilerParams(
            dimension_semantics=("parallel", "arbitrary"),
        ),
    )(query, key, value)
    return out
